# Initial kernel scaffold; baseline (speedup 1.0000x reference)
import jax
import jax.numpy as jnp
from jax.experimental import pallas as pl


def kernel(x, edge_index, edge_attr, emb, W, b):
    raise NotImplementedError("write your pallas kernel here")


# speedup vs baseline: 9.0156x; 9.0156x over previous
import functools

import jax
import jax.numpy as jnp
from jax import lax
from jax.experimental import pallas as pl
from jax.experimental.pallas import tpu as pltpu
from jax.experimental.pallas import tpu_sc as plsc

NC = 2
NS = 16
LANES = 16
CHUNK = 128
GCHUNK = 64
ROWBLK = 1024


def _round_up(v, m):
    return (v + m - 1) // m * m


def _make_gather_deg(F, npad, e2p, dw):
    cpt = e2p // (NC * NS * CHUNK)
    rpt = npad // (NC * NS)
    slab = npad // NS

    mesh = plsc.VectorSubcoreMesh(core_axis_name="c", subcore_axis_name="s")

    @functools.partial(
        pl.kernel,
        out_type=(
            jax.ShapeDtypeStruct((npad, F), jnp.float32),
            jax.ShapeDtypeStruct((NC, npad, dw), jnp.float32),
        ),
        mesh=mesh,
        scratch_types=[
            pltpu.VMEM((GCHUNK,), jnp.int32),
            pltpu.VMEM((GCHUNK, F), jnp.float32),
            pltpu.VMEM((CHUNK,), jnp.int32),
            pltpu.VMEM((CHUNK,), jnp.float32),
            pltpu.VMEM((CHUNK, dw), jnp.float32),
            pltpu.VMEM_SHARED((npad, dw), jnp.float32),
            pltpu.SemaphoreType.DMA,
        ],
    )
    def k1(emb_hbm, x_hbm, col_hbm, w_hbm, h0_hbm, deg_hbm,
           idx_v, grows_v, col_v, w_v, wsrc_v, deg_sh, sem):
        cid = lax.axis_index("c")
        sid = lax.axis_index("s")
        wid = cid * NS + sid

        def zrow(j, carry):
            for k in range(dw // LANES):
                wsrc_v[j, pl.ds(k * LANES, LANES)] = jnp.zeros((LANES,), jnp.float32)
            return carry
        lax.fori_loop(0, CHUNK, zrow, 0)
        for kk in range(slab // CHUNK):
            pltpu.sync_copy(wsrc_v, deg_sh.at[pl.ds(sid * slab + kk * CHUNK, CHUNK)])
        plsc.subcore_barrier()

        base = wid * rpt
        for i in range(rpt // GCHUNK):
            pltpu.sync_copy(x_hbm.at[pl.ds(base + i * GCHUNK, GCHUNK)], idx_v)
            pltpu.async_copy(emb_hbm.at[idx_v], grows_v, sem).wait()
            pltpu.sync_copy(grows_v, h0_hbm.at[pl.ds(base + i * GCHUNK, GCHUNK)])

        ebase = wid * cpt * CHUNK
        def deg_chunk(i, carry):
            off = ebase + i * CHUNK
            pltpu.sync_copy(col_hbm.at[pl.ds(off, CHUNK)], col_v)
            pltpu.sync_copy(w_hbm.at[pl.ds(off, CHUNK)], w_v)

            def fill(j, c2):
                w16 = w_v[pl.ds(j * LANES, LANES)]
                for l in range(LANES):
                    wb = jnp.broadcast_to(w16[l], (LANES,))
                    for k in range(dw // LANES):
                        wsrc_v[j * LANES + l, pl.ds(k * LANES, LANES)] = wb
                return c2
            lax.fori_loop(0, CHUNK // LANES, fill, 0)
            pltpu.sync_copy(wsrc_v, deg_sh.at[col_v], add=True)
            return carry
        lax.fori_loop(0, cpt, deg_chunk, 0)

        plsc.subcore_barrier()
        pltpu.sync_copy(deg_sh.at[pl.ds(sid * slab, slab)],
                        deg_hbm.at[cid, pl.ds(sid * slab, slab)])

    return k1


def _make_hop(F, npad, e2p):
    cpt = e2p // (NC * NS * CHUNK)
    slab = npad // NS
    nf16 = F // LANES

    mesh = plsc.VectorSubcoreMesh(core_axis_name="c", subcore_axis_name="s")

    @functools.partial(
        pl.kernel,
        out_type=jax.ShapeDtypeStruct((NC, npad, F), jnp.float32),
        mesh=mesh,
        scratch_types=[
            pltpu.VMEM((CHUNK,), jnp.int32),
            pltpu.VMEM((CHUNK,), jnp.int32),
            pltpu.VMEM((CHUNK,), jnp.float32),
            pltpu.VMEM((CHUNK, F), jnp.float32),
            pltpu.VMEM_SHARED((npad, F), jnp.float32),
            pltpu.SemaphoreType.DMA,
        ],
    )
    def hop(t_hbm, row_hbm, col_hbm, w_hbm, part_hbm,
            row_v, col_v, w_v, rows_v, acc_sh, sem):
        cid = lax.axis_index("c")
        sid = lax.axis_index("s")
        wid = cid * NS + sid

        def zrow(j, carry):
            for k in range(nf16):
                rows_v[j, pl.ds(k * LANES, LANES)] = jnp.zeros((LANES,), jnp.float32)
            return carry
        lax.fori_loop(0, CHUNK, zrow, 0)
        for kk in range(slab // CHUNK):
            pltpu.sync_copy(rows_v, acc_sh.at[pl.ds(sid * slab + kk * CHUNK, CHUNK)])
        plsc.subcore_barrier()

        ebase = wid * cpt * CHUNK
        def chunk_body(i, carry):
            off = ebase + i * CHUNK
            pltpu.sync_copy(row_hbm.at[pl.ds(off, CHUNK)], row_v)
            pltpu.sync_copy(col_hbm.at[pl.ds(off, CHUNK)], col_v)
            pltpu.sync_copy(w_hbm.at[pl.ds(off, CHUNK)], w_v)
            pltpu.async_copy(t_hbm.at[row_v], rows_v, sem).wait()

            def scale(j, c2):
                w16 = w_v[pl.ds(j * LANES, LANES)]
                for l in range(LANES):
                    e = j * LANES + l
                    s = w16[l]
                    for k in range(nf16):
                        rows_v[e, pl.ds(k * LANES, LANES)] = (
                            rows_v[e, pl.ds(k * LANES, LANES)] * s)
                return c2
            lax.fori_loop(0, CHUNK // LANES, scale, 0)
            pltpu.sync_copy(rows_v, acc_sh.at[col_v], add=True)
            return carry
        lax.fori_loop(0, cpt, chunk_body, 0)

        plsc.subcore_barrier()
        pltpu.sync_copy(acc_sh.at[pl.ds(sid * slab, slab)],
                        part_hbm.at[cid, pl.ds(sid * slab, slab)])

    return hop


def _prescale(deg, h0, npad, F, dw):
    grid = npad // ROWBLK

    def body(deg_ref, h0_ref, t0_ref, dis_ref):
        d = deg_ref[0] + deg_ref[1]
        degs = jnp.sum(d, axis=1, keepdims=True) * (1.0 / dw)
        dis = jnp.where(degs > 0, lax.rsqrt(degs), 0.0)
        disb = jnp.broadcast_to(dis, (ROWBLK, F))
        dis_ref[...] = disb
        t0_ref[...] = h0_ref[...] * disb

    return pl.pallas_call(
        body,
        grid=(grid,),
        in_specs=[
            pl.BlockSpec((NC, ROWBLK, dw), lambda i: (0, i, 0)),
            pl.BlockSpec((ROWBLK, F), lambda i: (i, 0)),
        ],
        out_specs=[
            pl.BlockSpec((ROWBLK, F), lambda i: (i, 0)),
            pl.BlockSpec((ROWBLK, F), lambda i: (i, 0)),
        ],
        out_shape=[
            jax.ShapeDtypeStruct((npad, F), jnp.float32),
            jax.ShapeDtypeStruct((npad, F), jnp.float32),
        ],
    )(deg, h0)


def _midscale(part, dis, npad, F):
    grid = npad // ROWBLK

    def body(p_ref, dis_ref, o_ref):
        d = dis_ref[...]
        o_ref[...] = (p_ref[0] + p_ref[1]) * d * d

    return pl.pallas_call(
        body,
        grid=(grid,),
        in_specs=[
            pl.BlockSpec((NC, ROWBLK, F), lambda i: (0, i, 0)),
            pl.BlockSpec((ROWBLK, F), lambda i: (i, 0)),
        ],
        out_specs=pl.BlockSpec((ROWBLK, F), lambda i: (i, 0)),
        out_shape=jax.ShapeDtypeStruct((npad, F), jnp.float32),
    )(part, dis)


def _final(part, dis, W, b2, n, npad, F, C):
    grid = npad // ROWBLK

    def body(p_ref, dis_ref, w_ref, b_ref, o_ref):
        h = (p_ref[0] + p_ref[1]) * dis_ref[...]
        z = jnp.dot(h, w_ref[...], preferred_element_type=jnp.float32)
        z = z + b_ref[...]
        m = jnp.max(z, axis=1, keepdims=True)
        e = jnp.exp(z - m)
        lse = jnp.log(jnp.sum(e, axis=1, keepdims=True)) + m
        o_ref[...] = z - lse

    return pl.pallas_call(
        body,
        grid=(grid,),
        in_specs=[
            pl.BlockSpec((NC, ROWBLK, F), lambda i: (0, i, 0)),
            pl.BlockSpec((ROWBLK, F), lambda i: (i, 0)),
            pl.BlockSpec((F, C), lambda i: (0, 0)),
            pl.BlockSpec((1, C), lambda i: (0, 0)),
        ],
        out_specs=pl.BlockSpec((ROWBLK, C), lambda i: (i, 0)),
        out_shape=jax.ShapeDtypeStruct((n, C), jnp.float32),
    )(part, dis, W, b2)


def kernel(x, edge_index, edge_attr, emb, W, b):
    n = x.shape[0]
    F = emb.shape[1]
    C = W.shape[1]
    E = edge_attr.shape[0]
    dw = F

    npad = _round_up(n, NC * NS * GCHUNK)
    e2 = E + n
    e2p = _round_up(e2, NC * NS * CHUNK)

    idt = jnp.int32
    loop_idx = jnp.arange(n, dtype=idt)
    row2 = jnp.concatenate([edge_index[0].astype(idt), loop_idx])
    col2 = jnp.concatenate([edge_index[1].astype(idt), loop_idx])
    w2 = jnp.concatenate([edge_attr.astype(jnp.float32),
                          jnp.ones((n,), jnp.float32)])
    pad = e2p - e2
    row2 = jnp.pad(row2, (0, pad))
    col2 = jnp.pad(col2, (0, pad))
    w2 = jnp.pad(w2, (0, pad))
    xp = jnp.pad(x.astype(idt), (0, npad - n))

    h0, deg = _make_gather_deg(F, npad, e2p, dw)(emb, xp, col2, w2)
    t0, dis = _prescale(deg, h0, npad, F, dw)
    hop = _make_hop(F, npad, e2p)
    p1 = hop(t0, row2, col2, w2)
    t1 = _midscale(p1, dis, npad, F)
    p2 = hop(t1, row2, col2, w2)
    return _final(p2, dis, W, b.reshape(1, C), n, npad, F, C)
